# baseline (speedup 1.0000x reference)
import jax
import jax.numpy as jnp
from jax import lax
from jax.experimental import pallas as pl
from jax.experimental.pallas import tpu as pltpu
from jax.experimental.pallas import tpu_sc as plsc

_EMBED_DIM = 32
_BATCH = 16384
_HIST = 200

_NW = 32
_BTG = 16
_UNITS = _HIST * _BTG
_PER_W = _UNITS // _NW
_PAD = 33


def _gather_body(lab5, tab, x, idx_v, rows_v, isem, gsem, xsem):
    w = lax.axis_index("s") * 2 + lax.axis_index("c")
    u0 = w * _PER_W

    def idx_start(u, d):
        h = u // _BTG
        btg = u % _BTG
        pltpu.async_copy(
            lab5.at[h // 8, pl.ds(btg * 8, 8), h % 8], idx_v.at[d], isem.at[d]
        )

    def idx_wait(d):
        pltpu.make_async_copy(
            lab5.at[0, pl.ds(0, 8), 0], idx_v.at[d], isem.at[d]
        ).wait()

    def gathers(d):
        for k in range(8):
            pltpu.async_copy(tab.at[idx_v.at[d, k]], rows_v.at[d, k], gsem.at[d])

    def gather_wait(d):
        for k in range(8):
            pltpu.make_async_copy(
                tab.at[idx_v.at[d, k]], rows_v.at[d, k], gsem.at[d]
            ).wait()

    def store_wait(d):
        pltpu.make_async_copy(rows_v.at[d], x.at[0], xsem.at[d]).wait()

    for d in range(2):
        idx_start(u0 + d, d)

    def outer(o, carry):
        for d in range(2):
            u = u0 + 2 * o + d
            idx_wait(d)
            if d == 0:
                @pl.when(o > 0)
                def _():
                    store_wait(d)
            else:
                @pl.when(o > 0)
                def _():
                    store_wait(d)
            gathers(d)
            gather_wait(d)
            pltpu.async_copy(rows_v.at[d], x.at[u], xsem.at[d])

            @pl.when(2 * o + d + 2 < _PER_W)
            def _():
                idx_start(u + 2, d)

        return carry

    lax.fori_loop(0, _PER_W // 2, outer, 0)
    for d in range(2):
        store_wait(d)


def _permute_body(x, out5, rows_p, obuf, lsem, ssem):
    w = lax.axis_index("s") * 2 + lax.axis_index("c")
    u0 = w * _PER_W

    def load_start(u, d):
        pltpu.async_copy(x.at[u], rows_p.at[d], lsem.at[d])

    def load_wait(d):
        pltpu.make_async_copy(x.at[0], rows_p.at[d], lsem.at[d]).wait()

    iota = lax.iota(jnp.int32, 16)
    et_vecs = [(half * 16 + iota) // 8 for half in range(2)]
    em_vecs = [(half * 16 + iota) % 8 for half in range(2)]

    def transpose(d):
        rows = rows_p.at[d]

        def per_k(k, carry):
            kvec = jnp.full((16,), k, jnp.int32)

            def per_bg(bg, c2):
                for t in range(16):
                    bm = bg * 16 + t
                    bmv = jnp.full((16,), bm, jnp.int32)
                    for half in range(2):
                        v = rows[k, bm, pl.ds(half * 16, 16)]
                        plsc.store_scatter(
                            obuf, [et_vecs[half], kvec, em_vecs[half], bmv], v
                        )
                return c2

            lax.fori_loop(0, 8, per_bg, 0, unroll=4)
            return carry

        lax.fori_loop(0, 8, per_k, 0)

    def store_start(u):
        h = u // _BTG
        btg = u % _BTG
        pltpu.async_copy(
            obuf.at[:, :, :, pl.ds(0, 128)], out5.at[h, :, pl.ds(btg * 8, 8)], ssem
        )

    def store_wait():
        pltpu.make_async_copy(
            out5.at[0, :, pl.ds(0, 8)], obuf.at[:, :, :, pl.ds(0, 128)], ssem
        ).wait()

    load_start(u0, 0)

    def outer(o, carry):
        for d in range(2):
            u = u0 + 2 * o + d

            @pl.when(2 * o + d + 1 < _PER_W)
            def _():
                load_start(u + 1, 1 - d)

            load_wait(d)
            if d == 0:
                @pl.when(o > 0)
                def _():
                    store_wait()
            else:
                store_wait()
            transpose(d)
            store_start(u)

        return carry

    lax.fori_loop(0, _PER_W // 2, outer, 0)
    store_wait()


@jax.jit
def _lookup(lab5, table):
    mesh = plsc.VectorSubcoreMesh(core_axis_name="c", subcore_axis_name="s")
    x = pl.kernel(
        _gather_body,
        out_type=jax.ShapeDtypeStruct((_UNITS, 8, 128, _EMBED_DIM), jnp.float32),
        mesh=mesh,
        scratch_types=[
            pltpu.VMEM((2, 8, 128), jnp.int32),
            pltpu.VMEM((2, 8, 128, _EMBED_DIM), jnp.float32),
            pltpu.SemaphoreType.DMA((2,)),
            pltpu.SemaphoreType.DMA((2,)),
            pltpu.SemaphoreType.DMA((2,)),
        ],
        compiler_params=pltpu.CompilerParams(use_tc_tiling_on_sc=False),
    )(lab5, table)
    out5 = pl.kernel(
        _permute_body,
        out_type=jax.ShapeDtypeStruct((_HIST, 4, 128, 8, 128), jnp.float32),
        mesh=mesh,
        scratch_types=[
            pltpu.VMEM((2, 8, 128, _EMBED_DIM), jnp.float32),
            pltpu.VMEM((4, 8, 8, 129), jnp.float32),
            pltpu.SemaphoreType.DMA((2,)),
            pltpu.SemaphoreType.DMA,
        ],
        compiler_params=pltpu.CompilerParams(
            use_tc_tiling_on_sc=False, needs_layout_passes=False
        ),
    )(x)
    return out5


def kernel(labels, embeddings):
    lab5 = labels.reshape(128, 128, 25, 8).transpose(2, 0, 3, 1)
    out5 = _lookup(lab5, embeddings)
    return out5.transpose(2, 4, 0, 1, 3).reshape(_BATCH, _HIST, _EMBED_DIM)


# speedup vs baseline: 1.0001x; 1.0001x over previous
import jax
import jax.numpy as jnp
from jax import lax
from jax.experimental import pallas as pl
from jax.experimental.pallas import tpu as pltpu
from jax.experimental.pallas import tpu_sc as plsc

_EMBED_DIM = 32
_BATCH = 16384
_HIST = 200

_NW = 32
_BTG = 16
_UNITS = _HIST * _BTG
_PER_W = _UNITS // _NW
_PAD = 33


def _gather_body(lab5, tab, x, idx_v, rows_v, isem, gsem, xsem):
    w = lax.axis_index("s") * 2 + lax.axis_index("c")
    u0 = w * _PER_W

    def idx_start(u, d):
        h = u // _BTG
        btg = u % _BTG
        pltpu.async_copy(
            lab5.at[h // 8, pl.ds(btg * 8, 8), h % 8], idx_v.at[d], isem.at[d]
        )

    def idx_wait(d):
        pltpu.make_async_copy(
            lab5.at[0, pl.ds(0, 8), 0], idx_v.at[d], isem.at[d]
        ).wait()

    def gathers(d):
        for k in range(8):
            pltpu.async_copy(tab.at[idx_v.at[d, k]], rows_v.at[d, k], gsem.at[d])

    def gather_wait(d):
        for k in range(8):
            pltpu.make_async_copy(
                tab.at[idx_v.at[d, k]], rows_v.at[d, k], gsem.at[d]
            ).wait()

    def store_wait(d):
        pltpu.make_async_copy(rows_v.at[d], x.at[0], xsem.at[d]).wait()

    for d in range(2):
        idx_start(u0 + d, d)

    def outer(o, carry):
        for d in range(2):
            u = u0 + 2 * o + d
            idx_wait(d)
            if d == 0:
                @pl.when(o > 0)
                def _():
                    store_wait(d)
            else:
                @pl.when(o > 0)
                def _():
                    store_wait(d)
            gathers(d)
            gather_wait(d)
            pltpu.async_copy(rows_v.at[d], x.at[u], xsem.at[d])

            @pl.when(2 * o + d + 2 < _PER_W)
            def _():
                idx_start(u + 2, d)

        return carry

    lax.fori_loop(0, _PER_W // 2, outer, 0)
    for d in range(2):
        store_wait(d)


def _permute_body(x, out5, rows_p, obuf, lsem, ssem):
    w = lax.axis_index("s") * 2 + lax.axis_index("c")
    u0 = w * _PER_W

    def load_start(u, d):
        pltpu.async_copy(x.at[u], rows_p.at[d], lsem.at[d])

    def load_wait(d):
        pltpu.make_async_copy(x.at[0], rows_p.at[d], lsem.at[d]).wait()

    iota = lax.iota(jnp.int32, 16)
    et_vecs = [(half * 16 + iota) // 8 for half in range(2)]
    em_vecs = [(half * 16 + iota) % 8 for half in range(2)]

    def transpose(d):
        rows = rows_p.at[d]

        def per_k(k, carry):
            kvec = jnp.full((16,), k, jnp.int32)

            def per_bg(bg, c2):
                for t in range(16):
                    bm = bg * 16 + t
                    bmv = jnp.full((16,), bm, jnp.int32)
                    for half in range(2):
                        v = rows[k, bm, pl.ds(half * 16, 16)]
                        plsc.store_scatter(
                            obuf, [et_vecs[half], kvec, em_vecs[half], bmv], v
                        )
                return c2

            lax.fori_loop(0, 8, per_bg, 0, unroll=4)
            return carry

        lax.fori_loop(0, 8, per_k, 0)

    def store_start(u):
        h = u // _BTG
        btg = u % _BTG
        pltpu.async_copy(
            obuf.at[:, :, :, pl.ds(0, 128)], out5.at[h, :, pl.ds(btg * 8, 8)], ssem
        )

    def store_wait():
        pltpu.make_async_copy(
            out5.at[0, :, pl.ds(0, 8)], obuf.at[:, :, :, pl.ds(0, 128)], ssem
        ).wait()

    load_start(u0, 0)

    def outer(o, carry):
        for d in range(2):
            u = u0 + 2 * o + d

            @pl.when(2 * o + d + 1 < _PER_W)
            def _():
                load_start(u + 1, 1 - d)

            load_wait(d)
            if d == 0:
                @pl.when(o > 0)
                def _():
                    store_wait()
            else:
                store_wait()
            transpose(d)
            store_start(u)

        return carry

    lax.fori_loop(0, _PER_W // 2, outer, 0)
    store_wait()


@jax.jit
def _lookup(lab5, table):
    mesh = plsc.VectorSubcoreMesh(core_axis_name="c", subcore_axis_name="s")
    x = pl.kernel(
        _gather_body,
        out_type=jax.ShapeDtypeStruct((_UNITS, 8, 128, _EMBED_DIM), jnp.float32),
        mesh=mesh,
        scratch_types=[
            pltpu.VMEM((2, 8, 128), jnp.int32),
            pltpu.VMEM((2, 8, 128, _EMBED_DIM), jnp.float32),
            pltpu.SemaphoreType.DMA((2,)),
            pltpu.SemaphoreType.DMA((2,)),
            pltpu.SemaphoreType.DMA((2,)),
        ],
        compiler_params=pltpu.CompilerParams(use_tc_tiling_on_sc=False),
    )(lab5, table)
    out5 = pl.kernel(
        _permute_body,
        out_type=jax.ShapeDtypeStruct((_HIST, 4, 128, 8, 128), jnp.float32),
        mesh=mesh,
        scratch_types=[
            pltpu.VMEM((2, 8, 128, _EMBED_DIM), jnp.float32),
            pltpu.VMEM((4, 8, 8, 131), jnp.float32),
            pltpu.SemaphoreType.DMA((2,)),
            pltpu.SemaphoreType.DMA,
        ],
        compiler_params=pltpu.CompilerParams(
            use_tc_tiling_on_sc=False, needs_layout_passes=False
        ),
    )(x)
    return out5


def kernel(labels, embeddings):
    lab5 = labels.reshape(128, 128, 25, 8).transpose(2, 0, 3, 1)
    out5 = _lookup(lab5, embeddings)
    return out5.transpose(2, 4, 0, 1, 3).reshape(_BATCH, _HIST, _EMBED_DIM)


# speedup vs baseline: 1.0003x; 1.0002x over previous
import jax
import jax.numpy as jnp
from jax import lax
from jax.experimental import pallas as pl
from jax.experimental.pallas import tpu as pltpu
from jax.experimental.pallas import tpu_sc as plsc

_EMBED_DIM = 32
_BATCH = 16384
_HIST = 200

_NW = 32
_BTG = 16
_UNITS = _HIST * _BTG
_PER_W = _UNITS // _NW
_OPAD = 131


def _gather_body(lab5, tab, x, idx_v, rows_v, isem, gsem, xsem):
    w = lax.axis_index("s") * 2 + lax.axis_index("c")
    u0 = w * _PER_W

    def idx_start(u, d):
        h = u // _BTG
        btg = u % _BTG
        pltpu.async_copy(
            lab5.at[h // 8, pl.ds(btg * 8, 8), h % 8], idx_v.at[d], isem.at[d]
        )

    def idx_wait(d):
        pltpu.make_async_copy(
            lab5.at[0, pl.ds(0, 8), 0], idx_v.at[d], isem.at[d]
        ).wait()

    def gathers(d):
        for k in range(8):
            pltpu.async_copy(tab.at[idx_v.at[d, k]], rows_v.at[d, k], gsem.at[d])

    def gather_wait(d):
        for k in range(8):
            pltpu.make_async_copy(
                tab.at[idx_v.at[d, k]], rows_v.at[d, k], gsem.at[d]
            ).wait()

    def store_wait(d):
        pltpu.make_async_copy(rows_v.at[d], x.at[0], xsem.at[d]).wait()

    for d in range(2):
        idx_start(u0 + d, d)

    def outer(o, carry):
        for d in range(2):
            u = u0 + 2 * o + d
            idx_wait(d)
            if d == 0:
                @pl.when(o > 0)
                def _():
                    store_wait(d)
            else:
                @pl.when(o > 0)
                def _():
                    store_wait(d)
            gathers(d)
            gather_wait(d)
            pltpu.async_copy(rows_v.at[d], x.at[u], xsem.at[d])

            @pl.when(2 * o + d + 2 < _PER_W)
            def _():
                idx_start(u + 2, d)

        return carry

    lax.fori_loop(0, _PER_W // 2, outer, 0)
    for d in range(2):
        store_wait(d)


def _permute_body(x, out5, rows_p, obuf, lsem, ssem):
    w = lax.axis_index("s") * 2 + lax.axis_index("c")
    u0 = w * _PER_W

    def load_start(u, d):
        pltpu.async_copy(x.at[u], rows_p.at[d], lsem.at[d])

    def load_wait(d):
        pltpu.make_async_copy(x.at[0], rows_p.at[d], lsem.at[d]).wait()

    iota = lax.iota(jnp.int32, 16)
    et_vecs = [(half * 16 + iota) // 8 for half in range(2)]
    em_vecs = [(half * 16 + iota) % 8 for half in range(2)]

    def transpose(d):
        rows = rows_p.at[d]

        def per_k(k, carry):
            kvec = jnp.full((16,), k, jnp.int32)

            def per_bg(bg, c2):
                for t in range(16):
                    bm = bg * 16 + t
                    bmv = jnp.full((16,), bm, jnp.int32)
                    for half in range(2):
                        v = rows[k, bm, pl.ds(half * 16, 16)]
                        plsc.store_scatter(
                            obuf, [et_vecs[half], kvec, em_vecs[half], bmv], v
                        )
                return c2

            lax.fori_loop(0, 8, per_bg, 0, unroll=4)
            return carry

        lax.fori_loop(0, 8, per_k, 0)

    def store_start(u):
        h = u // _BTG
        btg = u % _BTG
        pltpu.async_copy(
            obuf.at[:, :, :, pl.ds(0, 128)], out5.at[h, :, pl.ds(btg * 8, 8)], ssem
        )

    def store_wait():
        pltpu.make_async_copy(
            out5.at[0, :, pl.ds(0, 8)], obuf.at[:, :, :, pl.ds(0, 128)], ssem
        ).wait()

    load_start(u0, 0)

    def outer(o, carry):
        for d in range(2):
            u = u0 + 2 * o + d

            @pl.when(2 * o + d + 1 < _PER_W)
            def _():
                load_start(u + 1, 1 - d)

            load_wait(d)
            if d == 0:
                @pl.when(o > 0)
                def _():
                    store_wait()
            else:
                store_wait()
            transpose(d)
            store_start(u)

        return carry

    lax.fori_loop(0, _PER_W // 2, outer, 0)
    store_wait()


@jax.jit
def _lookup(lab5, table):
    mesh = plsc.VectorSubcoreMesh(core_axis_name="c", subcore_axis_name="s")
    x = pl.kernel(
        _gather_body,
        out_type=jax.ShapeDtypeStruct((_UNITS, 8, 128, _EMBED_DIM), jnp.float32),
        mesh=mesh,
        scratch_types=[
            pltpu.VMEM((2, 8, 128), jnp.int32),
            pltpu.VMEM((2, 8, 128, _EMBED_DIM), jnp.float32),
            pltpu.SemaphoreType.DMA((2,)),
            pltpu.SemaphoreType.DMA((2,)),
            pltpu.SemaphoreType.DMA((2,)),
        ],
        compiler_params=pltpu.CompilerParams(use_tc_tiling_on_sc=False),
    )(lab5, table)
    out5 = pl.kernel(
        _permute_body,
        out_type=jax.ShapeDtypeStruct((_HIST, 4, 128, 8, 128), jnp.float32),
        mesh=mesh,
        scratch_types=[
            pltpu.VMEM((2, 8, 128, _EMBED_DIM), jnp.float32),
            pltpu.VMEM((4, 8, 8, _OPAD), jnp.float32),
            pltpu.SemaphoreType.DMA((2,)),
            pltpu.SemaphoreType.DMA,
        ],
        compiler_params=pltpu.CompilerParams(
            use_tc_tiling_on_sc=False, needs_layout_passes=False
        ),
    )(x)
    return out5


def kernel(labels, embeddings):
    lab5 = labels.reshape(128, 128, 25, 8).transpose(2, 0, 3, 1)
    out5 = _lookup(lab5, embeddings)
    return out5.transpose(2, 4, 0, 1, 3).reshape(_BATCH, _HIST, _EMBED_DIM)
